# baseline (speedup 1.0000x reference)
import functools

import jax
import jax.numpy as jnp
from jax import lax
from jax.experimental import pallas as pl
from jax.experimental.pallas import tpu as pltpu
from jax.experimental.pallas import tpu_sc as plsc

NC = 2
NS = 16
NW = NC * NS
ROWS_PER_CHUNK = 2
LOOKAHEAD = 5
NBUF = 10


@functools.partial(jax.jit, static_argnames=("b1", "b2", "dim"))
def _gather_sc(x_flat, weight, b1, b2, dim):
    chunk = ROWS_PER_CHUNK * b2
    nchunk = (b1 * b2) // (NW * chunk)
    idx3 = x_flat.reshape(NW, nchunk * ROWS_PER_CHUNK, b2)
    mesh = plsc.VectorSubcoreMesh(
        core_axis_name="c", subcore_axis_name="s", num_cores=NC, num_subcores=NS
    )

    @functools.partial(
        pl.kernel,
        out_type=jax.ShapeDtypeStruct((b1, b2, dim), jnp.float32),
        mesh=mesh,
        scratch_types=[
            pltpu.VMEM((nchunk * ROWS_PER_CHUNK, b2), jnp.int32),
            pltpu.VMEM((NBUF, ROWS_PER_CHUNK, b2, dim), jnp.float32),
            pltpu.SemaphoreType.DMA((NBUF,)),
            pltpu.SemaphoreType.DMA((NBUF,)),
        ],
        compiler_params=pltpu.CompilerParams(use_tc_tiling_on_sc=False),
    )
    def k(idx_hbm, table_hbm, out_hbm, idx_v, rows_v, gsem, wsem):
        wid = lax.axis_index("s") * NC + lax.axis_index("c")
        pltpu.sync_copy(idx_hbm.at[wid], idx_v)
        row_base = wid * (nchunk * ROWS_PER_CHUNK)

        def issue_gather(c):
            b = c % NBUF
            return [
                pltpu.async_copy(
                    table_hbm.at[idx_v.at[c * ROWS_PER_CHUNK + r]],
                    rows_v.at[b, r],
                    gsem.at[b],
                )
                for r in range(ROWS_PER_CHUNK)
            ]

        def issue_write(c):
            b = c % NBUF
            dst = out_hbm.at[pl.ds(row_base + c * ROWS_PER_CHUNK, ROWS_PER_CHUNK)]
            return pltpu.async_copy(rows_v.at[b], dst, wsem.at[b])

        gathers, writes = {}, {}
        for c in range(min(LOOKAHEAD, nchunk)):
            gathers[c] = issue_gather(c)
        for j in range(nchunk):
            f = j + LOOKAHEAD
            if f < nchunk:
                if f >= NBUF:
                    writes[f - NBUF].wait()
                gathers[f] = issue_gather(f)
            for g in gathers[j]:
                g.wait()
            writes[j] = issue_write(j)
        for j in range(max(0, nchunk - NBUF), nchunk):
            writes[j].wait()

    return k(idx3, weight)


def kernel(x, weight):
    dim = weight.shape[1]
    b1, b2 = x.shape
    return _gather_sc(x.reshape(-1), weight, b1, b2, dim)
